# baseline (speedup 1.0000x reference)
import jax
import jax.numpy as jnp
from jax.experimental import pallas as pl
from jax.experimental.pallas import tpu as pltpu

_VOCAB = 1000000
_DIM = 64
_CTX = 200
_BV = 8000
_NB = _VOCAB // _BV


def _gather_sum_kernel(idx_ref, row_ref, acc_ref):
    @pl.when(pl.program_id(0) == 0)
    def _():
        acc_ref[...] = jnp.zeros_like(acc_ref)

    acc_ref[...] += row_ref[...]


def _scores_kernel(sum_ref, w_ref, b_ref, out_ref, lse_ref, stat_ref):
    i = pl.program_id(0)

    @pl.when(i == 0)
    def _():
        stat_ref[0] = -jnp.inf
        stat_ref[1] = 0.0

    m = sum_ref[...] * (1.0 / _CTX)
    s = jax.lax.dot_general(
        m, w_ref[...], (((1,), (1,)), ((), ())),
        preferred_element_type=jnp.float32,
    ) + b_ref[0]
    out_ref[...] = s[None]

    old_max = stat_ref[0]
    new_max = jnp.maximum(old_max, jnp.max(s))
    stat_ref[1] = stat_ref[1] * jnp.exp(old_max - new_max) + jnp.sum(
        jnp.exp(s - new_max))
    stat_ref[0] = new_max

    @pl.when(i == _NB - 1)
    def _():
        lse_ref[0] = stat_ref[0] + jnp.log(stat_ref[1])


def _normalize_kernel(lse_ref, raw_ref, out_ref):
    out_ref[...] = raw_ref[...] - lse_ref[0]


@jax.jit
def kernel(inputs, emb_table, W, b):
    idx = inputs.astype(jnp.int32)

    row_sum = pl.pallas_call(
        _gather_sum_kernel,
        grid_spec=pltpu.PrefetchScalarGridSpec(
            num_scalar_prefetch=1,
            grid=(_CTX,),
            in_specs=[
                pl.BlockSpec((1, 1, _DIM), lambda i, idx_ref: (idx_ref[i], 0, 0)),
            ],
            out_specs=pl.BlockSpec((1, 1, _DIM), lambda i, idx_ref: (0, 0, 0)),
        ),
        out_shape=jax.ShapeDtypeStruct((1, 1, _DIM), jnp.float32),
    )(idx, emb_table.reshape(_VOCAB, 1, _DIM))

    raw, lse = pl.pallas_call(
        _scores_kernel,
        grid=(_NB,),
        in_specs=[
            pl.BlockSpec((1, _DIM), lambda i: (0, 0)),
            pl.BlockSpec((_BV, _DIM), lambda i: (i, 0)),
            pl.BlockSpec((1, 1, _BV), lambda i: (i, 0, 0)),
        ],
        out_specs=[
            pl.BlockSpec((1, 1, _BV), lambda i: (i, 0, 0)),
            pl.BlockSpec(memory_space=pltpu.SMEM),
        ],
        out_shape=[
            jax.ShapeDtypeStruct((_NB, 1, _BV), jnp.float32),
            jax.ShapeDtypeStruct((1,), jnp.float32),
        ],
        scratch_shapes=[pltpu.SMEM((2,), jnp.float32)],
    )(row_sum.reshape(1, _DIM), W, b.reshape(_NB, 1, _BV))

    log_probs = pl.pallas_call(
        _normalize_kernel,
        grid=(_NB,),
        in_specs=[
            pl.BlockSpec(memory_space=pltpu.SMEM),
            pl.BlockSpec((1, 1, _BV), lambda i: (i, 0, 0)),
        ],
        out_specs=pl.BlockSpec((1, 1, _BV), lambda i: (i, 0, 0)),
        out_shape=jax.ShapeDtypeStruct((_NB, 1, _BV), jnp.float32),
    )(lse, raw)

    return log_probs.reshape(1, _VOCAB)
